# baseline (speedup 1.0000x reference)
import jax
import jax.numpy as jnp
from jax.experimental import pallas as pl
from jax.experimental.pallas import tpu as pltpu

_N, _D = 1_000_000, 3
_HEAD = 8


def _dma_body(x_ref, o_ref, patch, sem_p, sem_b):
    pltpu.make_async_copy(x_ref.at[pl.ds(0, _HEAD)], patch, sem_p).start()
    pltpu.make_async_copy(x_ref, o_ref, sem_b).start()

    pltpu.make_async_copy(x_ref.at[pl.ds(0, _HEAD)], patch, sem_p).wait()
    r = jax.lax.broadcasted_iota(jnp.int32, (_HEAD, _D), 0)
    c = jax.lax.broadcasted_iota(jnp.int32, (_HEAD, _D), 1).astype(jnp.float32)
    vals = patch[...]
    patch[...] = jnp.where(r == 0, 10.0 + c, jnp.where(r == 2, 20.0 + c, vals))

    pltpu.make_async_copy(x_ref, o_ref, sem_b).wait()
    pltpu.make_async_copy(patch, o_ref.at[pl.ds(0, _HEAD)], sem_p).start()
    pltpu.make_async_copy(patch, o_ref.at[pl.ds(0, _HEAD)], sem_p).wait()


def kernel(x):
    return pl.pallas_call(
        _dma_body,
        in_specs=[pl.BlockSpec(memory_space=pl.ANY)],
        out_specs=pl.BlockSpec(memory_space=pl.ANY),
        out_shape=jax.ShapeDtypeStruct((_N, _D), jnp.float32),
        scratch_shapes=[
            pltpu.VMEM((_HEAD, _D), jnp.float32),
            pltpu.SemaphoreType.DMA,
            pltpu.SemaphoreType.DMA,
        ],
    )(x)


# speedup vs baseline: 4.0597x; 4.0597x over previous
import jax
import jax.numpy as jnp
import numpy as np
from jax.experimental import pallas as pl

_N, _D = 1_000_000, 3
_F = _N * _D
_BLK = 524288
_GRID = -(-_F // _BLK)

def _copy_body(x_ref, o_ref):
    o_ref[...] = x_ref[...]

    @pl.when(pl.program_id(0) == 0)
    def _():
        i = jax.lax.broadcasted_iota(jnp.int32, (128,), 0)
        f = i.astype(jnp.float32)
        vals = x_ref[0:128]
        vals = jnp.where(i < 3, 10.0 + f, vals)
        vals = jnp.where((i >= 6) & (i < 9), 14.0 + f, vals)
        o_ref[0:128] = vals


def kernel(x):
    xf = jnp.reshape(x, (_F,))
    out = pl.pallas_call(
        _copy_body,
        grid=(_GRID,),
        in_specs=[pl.BlockSpec((_BLK,), lambda i: (i,))],
        out_specs=pl.BlockSpec((_BLK,), lambda i: (i,)),
        out_shape=jax.ShapeDtypeStruct((_F,), jnp.float32),
    )(xf)
    return jnp.reshape(out, (_N, _D))
